# Initial kernel scaffold; baseline (speedup 1.0000x reference)
import jax
import jax.numpy as jnp
from jax.experimental import pallas as pl


def kernel(x, edge_index, edge_attr, W1, We1, b1, W2, We2, b2):
    raise NotImplementedError("write your pallas kernel here")


# speedup vs baseline: 5.2070x; 5.2070x over previous
import functools

import jax
import jax.numpy as jnp
from jax import lax
from jax.experimental import pallas as pl
from jax.experimental.pallas import tpu as pltpu
from jax.experimental.pallas import tpu_sc as plsc

N_NODES = 10000
N_EDGES = 320000
D_FEAT = 128
D_EDGE = 16

NC = 2
NS = 16
NW = NC * NS

EPW = N_EDGES // NW
CK = 80
OC = 400
UI = OC // CK
NOUT = EPW // OC
RPT = N_NODES // NS
ZR = 25
ZR16 = 125


def _sc_body(x_hbm, src_hbm, dst_hbm, ea_hbm, z128_hbm, z16_hbm, one_hbm,
             a_out, b_out, d_out,
             srcb, dstb, ebuf, rows, onesb, zb,
             acc_a, acc_b, acc_d, gsem):
    cid = lax.axis_index("c")
    sid = lax.axis_index("s")
    wid = cid * NS + sid

    pltpu.sync_copy(z128_hbm, zb)
    pltpu.sync_copy(z16_hbm, ebuf.at[pl.ds(0, ZR16)])

    @pl.loop(0, RPT // ZR)
    def _zero_a(j):
        pltpu.sync_copy(zb, acc_a.at[pl.ds(sid * RPT + j * ZR, ZR)])

    @pl.loop(0, RPT // ZR16)
    def _zero_bd(j):
        r0 = sid * RPT + j * ZR16
        pltpu.sync_copy(ebuf.at[pl.ds(0, ZR16)], acc_b.at[pl.ds(r0, ZR16)])
        pltpu.sync_copy(ebuf.at[pl.ds(0, ZR16)], acc_d.at[pl.ds(r0, ZR16)])

    pltpu.sync_copy(one_hbm, onesb)
    plsc.subcore_barrier()

    @pl.loop(0, NOUT)
    def _outer(c):
        ebase = wid * EPW + c * OC
        pltpu.sync_copy(src_hbm.at[pl.ds(ebase, OC)], srcb)
        pltpu.sync_copy(dst_hbm.at[pl.ds(wid * (EPW // CK) + c * UI, UI)], dstb)
        pltpu.sync_copy(ea_hbm.at[pl.ds(ebase, OC)], ebuf)

        @pl.loop(0, UI)
        def _inner(u):
            pltpu.async_copy(
                x_hbm.at[srcb.at[pl.ds(u * CK, CK)]], rows, gsem).wait()
            pltpu.sync_copy(rows, acc_a.at[dstb.at[u]], add=True)
            pltpu.sync_copy(ebuf.at[pl.ds(u * CK, CK)],
                            acc_b.at[dstb.at[u]], add=True)
            pltpu.sync_copy(onesb, acc_d.at[dstb.at[u]], add=True)

    plsc.subcore_barrier()

    @pl.loop(0, RPT // ZR)
    def _out_a(j):
        r0 = sid * RPT + j * ZR
        pltpu.sync_copy(acc_a.at[pl.ds(r0, ZR)], zb)
        pltpu.sync_copy(zb, a_out.at[cid].at[pl.ds(r0, ZR)])

    @pl.loop(0, RPT // ZR16)
    def _out_bd(j):
        r0 = sid * RPT + j * ZR16
        pltpu.sync_copy(acc_b.at[pl.ds(r0, ZR16)], ebuf.at[pl.ds(0, ZR16)])
        pltpu.sync_copy(ebuf.at[pl.ds(0, ZR16)],
                        b_out.at[cid].at[pl.ds(r0, ZR16)])
        pltpu.sync_copy(acc_d.at[pl.ds(r0, ZR16)], ebuf.at[pl.ds(0, ZR16)])
        pltpu.sync_copy(ebuf.at[pl.ds(0, ZR16)],
                        d_out.at[cid].at[pl.ds(r0, ZR16)])


def _sc_segment_sums(x, src, dst2d, edge_attr, z128, z16, ones16):
    mesh = plsc.VectorSubcoreMesh(core_axis_name="c", subcore_axis_name="s",
                                  num_cores=NC, num_subcores=NS)
    f32 = jnp.float32
    return pl.kernel(
        _sc_body,
        out_type=(
            jax.ShapeDtypeStruct((NC, N_NODES, D_FEAT), f32),
            jax.ShapeDtypeStruct((NC, N_NODES, D_EDGE), f32),
            jax.ShapeDtypeStruct((NC, N_NODES, D_EDGE), f32),
        ),
        mesh=mesh,
        scratch_types=[
            pltpu.VMEM((OC,), jnp.int32),
            pltpu.VMEM((UI, CK), jnp.int32),
            pltpu.VMEM((OC, D_EDGE), f32),
            pltpu.VMEM((CK, D_FEAT), f32),
            pltpu.VMEM((CK, D_EDGE), f32),
            pltpu.VMEM((ZR, D_FEAT), f32),
            pltpu.VMEM_SHARED((N_NODES, D_FEAT), f32),
            pltpu.VMEM_SHARED((N_NODES, D_EDGE), f32),
            pltpu.VMEM_SHARED((N_NODES, D_EDGE), f32),
            pltpu.SemaphoreType.DMA,
        ],
        compiler_params=pltpu.CompilerParams(use_tc_tiling_on_sc=False),
    )(x, src, dst2d, edge_attr, z128, z16, ones16)


ROWS_BLK = 1000


def _finish_body(a_ref, b_ref, d_ref, w_ref, we_ref, bias_ref, o_ref):
    a = a_ref[0] + a_ref[1]
    b = b_ref[0] + b_ref[1]
    deg = jnp.maximum(d_ref[0, :, 0:1] + d_ref[1, :, 0:1], 1.0)
    agg = jnp.dot(a, w_ref[...], preferred_element_type=jnp.float32)
    agg = agg + jnp.dot(b, we_ref[...], preferred_element_type=jnp.float32)
    v = agg / deg + bias_ref[...]
    o_ref[...] = jnp.where(v > 0, v, jnp.exp(v) - 1.0)


def _tc_finish(a_parts, b_parts, d_parts, W, We, bias):
    grid = (N_NODES // ROWS_BLK,)
    return pl.pallas_call(
        _finish_body,
        out_shape=jax.ShapeDtypeStruct((N_NODES, D_FEAT), jnp.float32),
        grid=grid,
        in_specs=[
            pl.BlockSpec((NC, ROWS_BLK, D_FEAT), lambda i: (0, i, 0)),
            pl.BlockSpec((NC, ROWS_BLK, D_EDGE), lambda i: (0, i, 0)),
            pl.BlockSpec((NC, ROWS_BLK, D_EDGE), lambda i: (0, i, 0)),
            pl.BlockSpec((D_FEAT, D_FEAT), lambda i: (0, 0)),
            pl.BlockSpec((D_EDGE, D_FEAT), lambda i: (0, 0)),
            pl.BlockSpec((1, D_FEAT), lambda i: (0, 0)),
        ],
        out_specs=pl.BlockSpec((ROWS_BLK, D_FEAT), lambda i: (i, 0)),
    )(a_parts, b_parts, d_parts, W, We, bias)


@jax.jit
def kernel(x, edge_index, edge_attr, W1, We1, b1, W2, We2, b2):
    src = edge_index[0].astype(jnp.int32)
    dst = edge_index[1].astype(jnp.int32)
    dst2d = dst.reshape(N_EDGES // CK, CK)
    z128 = jnp.zeros((ZR, D_FEAT), jnp.float32)
    z16 = jnp.zeros((ZR16, D_EDGE), jnp.float32)
    ones16 = jnp.ones((CK, D_EDGE), jnp.float32)
    a_parts, b_parts, d_parts = _sc_segment_sums(
        x, src, dst2d, edge_attr, z128, z16, ones16)
    return _tc_finish(a_parts, b_parts, d_parts, W2, We2,
                      b2.reshape(1, D_FEAT))
